# baseline (speedup 1.0000x reference)
import functools

import jax
import jax.numpy as jnp
from jax import lax
from jax.experimental import pallas as pl
from jax.experimental.pallas import tpu as pltpu
from jax.experimental.pallas import tpu_sc as plsc

ENT_NUM = 1000000
REL_NUM = 1000
HIDDEN = 32
B = 16384
SR = 64
NSR = ENT_NUM // 2
NC = 2
NS = 16
L = 16
NW = NC * NS
BPW = B // NW
G = BPW // L
CHUNK = 128
NCHK = BPW // CHUNK


def _tec_kernel(p_h, p_t, p_r, n_h, n_t, n_r, ent, rel,
                p_out, n_out,
                ih, it, ir, th, tt,
                stg_h, stg_t, stg_r, score, sem_g):
    wid = lax.axis_index("s") * NC + lax.axis_index("c")
    base = wid * BPW

    lane = lax.iota(jnp.int32, L)

    def _phase(h_hbm, t_hbm, r_hbm, out_hbm):
        pltpu.sync_copy(h_hbm.at[pl.ds(base, BPW)], ih)
        pltpu.sync_copy(t_hbm.at[pl.ds(base, BPW)], it)
        pltpu.sync_copy(r_hbm.at[pl.ds(base, BPW)], ir)

        def tbody(i, carry):
            sl = pl.ds(i * L, L)
            th[sl] = lax.shift_right_logical(ih[sl], 1)
            tt[sl] = lax.shift_right_logical(it[sl], 1)
            return carry
        lax.fori_loop(0, G, tbody, 0)

        copies = []
        for c in range(NCHK):
            sl = pl.ds(c * CHUNK, CHUNK)
            copies.append(pltpu.async_copy(ent.at[th.at[sl]], stg_h.at[sl], sem_g))
            copies.append(pltpu.async_copy(ent.at[tt.at[sl]], stg_t.at[sl], sem_g))
            copies.append(pltpu.async_copy(rel.at[ir.at[sl]], stg_r.at[sl], sem_g))
        for cp in copies:
            cp.wait()

        def gbody(g, carry):
            sl = pl.ds(g * L, L)
            slot = g * L + lane
            cb_h = jnp.bitwise_and(ih[sl], 1) * HIDDEN
            cb_t = jnp.bitwise_and(it[sl], 1) * HIDDEN
            acc = jnp.zeros((L,), jnp.float32)
            for j in range(HIDDEN):
                col = jnp.bitwise_and(lane + j, HIDDEN - 1)
                hv = plsc.load_gather(stg_h, [slot, cb_h + col])
                tv = plsc.load_gather(stg_t, [slot, cb_t + col])
                rv = plsc.load_gather(stg_r, [slot, col])
                acc = acc + jnp.abs(hv + rv - tv)
            score[sl] = acc
            return carry
        lax.fori_loop(0, G, gbody, 0)
        pltpu.sync_copy(score, out_hbm.at[pl.ds(base, BPW)])

    _phase(p_h, p_t, p_r, p_out)
    _phase(n_h, n_t, n_r, n_out)


@jax.jit
def kernel(p_h, p_t, p_r, n_h, n_t, n_r, ent_emb, rel_emb):
    ent2 = ent_emb.reshape(NSR, SR)
    mesh = plsc.VectorSubcoreMesh(core_axis_name="c", subcore_axis_name="s")
    f32 = jnp.float32
    i32 = jnp.int32
    run = pl.kernel(
        _tec_kernel,
        out_type=(jax.ShapeDtypeStruct((B,), f32),
                  jax.ShapeDtypeStruct((B,), f32)),
        mesh=mesh,
        scratch_types=(
            [pltpu.VMEM((BPW,), i32) for _ in range(5)]
            + [pltpu.VMEM((BPW, SR), f32) for _ in range(2)]
            + [pltpu.VMEM((BPW, HIDDEN), f32)]
            + [pltpu.VMEM((BPW,), f32)]
            + [pltpu.SemaphoreType.DMA]
        ),
        compiler_params=pltpu.CompilerParams(
            needs_layout_passes=False, use_tc_tiling_on_sc=False),
    )
    return run(p_h, p_t, p_r, n_h, n_t, n_r, ent2, rel_emb)


# speedup vs baseline: 1.0029x; 1.0029x over previous
import functools

import jax
import jax.numpy as jnp
from jax import lax
from jax.experimental import pallas as pl
from jax.experimental.pallas import tpu as pltpu
from jax.experimental.pallas import tpu_sc as plsc

ENT_NUM = 1000000
REL_NUM = 1000
HIDDEN = 32
PADW = 128
B = 16384
NC = 2
NS = 16
L = 16
NW = NC * NS
BPW = B // NW
CH = 256
NR = BPW // CH
CHUNK = 128
GPR = CH // L


def _tec_kernel(p_h, p_t, p_r, n_h, n_t, n_r, ent, rel,
                p_out, n_out,
                ih, it, ir, stg_h, stg_t, stg_r, score, sem_g):
    wid = lax.axis_index("s") * NC + lax.axis_index("c")
    base = wid * BPW

    lane = lax.iota(jnp.int32, L)

    def _phase(h_hbm, t_hbm, r_hbm, out_hbm):
        pltpu.sync_copy(h_hbm.at[pl.ds(base, BPW)], ih)
        pltpu.sync_copy(t_hbm.at[pl.ds(base, BPW)], it)
        pltpu.sync_copy(r_hbm.at[pl.ds(base, BPW)], ir)

        def round_(r):
            copies = []
            for c in range(CH // CHUNK):
                isl = pl.ds(r * CH + c * CHUNK, CHUNK)
                ssl = pl.ds(c * CHUNK, CHUNK)
                copies.append(
                    pltpu.async_copy(ent.at[ih.at[isl]], stg_h.at[ssl], sem_g))
                copies.append(
                    pltpu.async_copy(ent.at[it.at[isl]], stg_t.at[ssl], sem_g))
                copies.append(
                    pltpu.async_copy(rel.at[ir.at[isl]], stg_r.at[ssl], sem_g))
            for cp in copies:
                cp.wait()

            def gbody(g, carry):
                slot = g * L + lane
                acc = jnp.zeros((L,), jnp.float32)
                for j in range(HIDDEN):
                    col = jnp.bitwise_and(lane + j, HIDDEN - 1)
                    hv = plsc.load_gather(stg_h, [slot, col])
                    tv = plsc.load_gather(stg_t, [slot, col])
                    rv = plsc.load_gather(stg_r, [slot, col])
                    acc = acc + jnp.abs(hv + rv - tv)
                score[pl.ds(r * CH + g * L, L)] = acc
                return carry
            lax.fori_loop(0, GPR, gbody, 0)

        for r in range(NR):
            round_(r)
        pltpu.sync_copy(score, out_hbm.at[pl.ds(base, BPW)])

    _phase(p_h, p_t, p_r, p_out)
    _phase(n_h, n_t, n_r, n_out)


@jax.jit
def kernel(p_h, p_t, p_r, n_h, n_t, n_r, ent_emb, rel_emb):
    ent_pad = jnp.pad(ent_emb, ((0, 0), (0, PADW - HIDDEN)))
    rel_pad = jnp.pad(rel_emb, ((0, 0), (0, PADW - HIDDEN)))
    mesh = plsc.VectorSubcoreMesh(core_axis_name="c", subcore_axis_name="s")
    f32 = jnp.float32
    i32 = jnp.int32
    run = pl.kernel(
        _tec_kernel,
        out_type=(jax.ShapeDtypeStruct((B,), f32),
                  jax.ShapeDtypeStruct((B,), f32)),
        mesh=mesh,
        scratch_types=(
            [pltpu.VMEM((BPW,), i32) for _ in range(3)]
            + [pltpu.VMEM((CH, PADW), f32) for _ in range(3)]
            + [pltpu.VMEM((BPW,), f32)]
            + [pltpu.SemaphoreType.DMA]
        ),
        compiler_params=pltpu.CompilerParams(needs_layout_passes=False),
    )
    return run(p_h, p_t, p_r, n_h, n_t, n_r, ent_pad, rel_pad)


# speedup vs baseline: 1.0137x; 1.0107x over previous
import functools

import jax
import jax.numpy as jnp
from jax import lax
from jax.experimental import pallas as pl
from jax.experimental.pallas import tpu as pltpu
from jax.experimental.pallas import tpu_sc as plsc

HIDDEN = 32
B = 16384
NC = 2
NS = 16
L = 16
NW = NC * NS
BPW = B // NW
G = BPW // L
CHUNK = 128
NCH = BPW // CHUNK


def _tec_kernel(p_h, p_t, p_r, n_h, n_t, n_r, ent, rel,
                p_out, n_out,
                iph, ipt, ipr, inh, int_, inr,
                rph, rpt, rpr, rnh, rnt, rnr,
                score_p, score_n, sem_p, sem_n):
    wid = lax.axis_index("s") * NC + lax.axis_index("c")
    base = wid * BPW

    pltpu.sync_copy(p_h.at[pl.ds(base, BPW)], iph)
    pltpu.sync_copy(p_t.at[pl.ds(base, BPW)], ipt)
    pltpu.sync_copy(p_r.at[pl.ds(base, BPW)], ipr)
    pltpu.sync_copy(n_h.at[pl.ds(base, BPW)], inh)
    pltpu.sync_copy(n_t.at[pl.ds(base, BPW)], int_)
    pltpu.sync_copy(n_r.at[pl.ds(base, BPW)], inr)

    copies_p = []
    copies_n = []
    for c in range(NCH):
        sl = pl.ds(c * CHUNK, CHUNK)
        copies_p.append(pltpu.async_copy(ent.at[iph.at[sl]], rph.at[sl], sem_p))
        copies_p.append(pltpu.async_copy(ent.at[ipt.at[sl]], rpt.at[sl], sem_p))
        copies_p.append(pltpu.async_copy(rel.at[ipr.at[sl]], rpr.at[sl], sem_p))
    for c in range(NCH):
        sl = pl.ds(c * CHUNK, CHUNK)
        copies_n.append(pltpu.async_copy(ent.at[inh.at[sl]], rnh.at[sl], sem_n))
        copies_n.append(pltpu.async_copy(ent.at[int_.at[sl]], rnt.at[sl], sem_n))
        copies_n.append(pltpu.async_copy(rel.at[inr.at[sl]], rnr.at[sl], sem_n))

    lane = lax.iota(jnp.int32, L)

    def _reduce(rh, rt, rr, score):
        def gbody(g, carry):
            row = g * L + lane
            acc = jnp.zeros((L,), jnp.float32)
            for j in range(HIDDEN):
                col = jnp.bitwise_and(lane + j, HIDDEN - 1)
                hv = plsc.load_gather(rh, [row, col])
                tv = plsc.load_gather(rt, [row, col])
                rv = plsc.load_gather(rr, [row, col])
                acc = acc + jnp.abs(hv + rv - tv)
            score[pl.ds(g * L, L)] = acc
            return carry
        lax.fori_loop(0, G, gbody, 0)

    for cp in copies_p:
        cp.wait()
    _reduce(rph, rpt, rpr, score_p)
    pltpu.sync_copy(score_p, p_out.at[pl.ds(base, BPW)])

    for cp in copies_n:
        cp.wait()
    _reduce(rnh, rnt, rnr, score_n)
    pltpu.sync_copy(score_n, n_out.at[pl.ds(base, BPW)])


@jax.jit
def kernel(p_h, p_t, p_r, n_h, n_t, n_r, ent_emb, rel_emb):
    mesh = plsc.VectorSubcoreMesh(core_axis_name="c", subcore_axis_name="s")
    f32 = jnp.float32
    i32 = jnp.int32
    run = pl.kernel(
        _tec_kernel,
        out_type=(jax.ShapeDtypeStruct((B,), f32),
                  jax.ShapeDtypeStruct((B,), f32)),
        mesh=mesh,
        scratch_types=(
            [pltpu.VMEM((BPW,), i32) for _ in range(6)]
            + [pltpu.VMEM((BPW, HIDDEN), f32) for _ in range(6)]
            + [pltpu.VMEM((BPW,), f32) for _ in range(2)]
            + [pltpu.SemaphoreType.DMA, pltpu.SemaphoreType.DMA]
        ),
        compiler_params=pltpu.CompilerParams(
            needs_layout_passes=False, use_tc_tiling_on_sc=False),
    )
    return run(p_h, p_t, p_r, n_h, n_t, n_r, ent_emb, rel_emb)
